# baseline (speedup 1.0000x reference)
import functools

import jax
import jax.numpy as jnp
from jax import lax
from jax.experimental import pallas as pl
from jax.experimental.pallas import tpu as pltpu
from jax.experimental.pallas import tpu_sc as plsc

_P = 7
_S = 2
_SCALE = 0.25
_L = 16
_NC = 2
_NS = 16
_NW = _NC * _NS
_KQ = _P * _S
_NG = 8
_NU = _NG * _KQ
_NR = 128


def _bf16x2(w):
    wi = plsc.bitcast(w, jnp.int32)
    wi = (wi + jnp.int32(0x8000)) & jnp.int32(-0x10000)
    pair = wi | lax.shift_right_logical(wi, 16)
    return plsc.bitcast(pair, jnp.bfloat16)


def _take(vec, i):
    idx = jnp.full((_L, 1), i, jnp.int32)
    dnums = lax.GatherDimensionNumbers(
        offset_dims=(), collapsed_slice_dims=(0,), start_index_map=(0,))
    return lax.gather(vec, idx, dnums, (1,),
                      mode=lax.GatherScatterMode.PROMISE_IN_BOUNDS)


@functools.lru_cache(maxsize=None)
def _make_roi_kernel(N, C, H, W, M):
    BPW = M // _NW
    PP = _P * _P
    CW = C // 2
    mesh = plsc.VectorSubcoreMesh(core_axis_name="c", subcore_axis_name="s")

    @functools.partial(
        pl.kernel,
        mesh=mesh,
        compiler_params=pltpu.CompilerParams(needs_layout_passes=False),
        out_type=jax.ShapeDtypeStruct((M, PP, CW), jnp.int32),
        scratch_types=[
            pltpu.VMEM((5 * BPW,), jnp.float32),
            pltpu.VMEM((4, _L), jnp.float32),
            pltpu.VMEM((4, _L), jnp.int32),
            pltpu.VMEM((2, _NR), jnp.int32),
            pltpu.VMEM((2, _NR, CW), jnp.int32),
            pltpu.VMEM((2, 56, CW), jnp.int32),
            pltpu.SemaphoreType.DMA,
            pltpu.SemaphoreType.DMA,
        ],
    )
    def roi_sc(table_hbm, boxes_hbm, out_hbm, boxes_v, pf_v, pi_v, idx_v, g_v,
               o_v, gsem, osem):
        wid = lax.axis_index("s") * _NC + lax.axis_index("c")
        for f in range(5):
            pltpu.sync_copy(boxes_hbm.at[pl.ds(f * M + wid * BPW, BPW)],
                            boxes_v.at[pl.ds(f * BPW, BPW)])

        iota = lax.iota(jnp.int32, _L)
        lane_mask = iota < _KQ
        for par0 in range(2):
            idx_v[par0, pl.ds(_NU, _L)] = jnp.zeros((_L,), jnp.int32)

        def compute_params(bi):
            chunk = (bi // _L) * _L
            lane = bi - chunk

            def field(f):
                return _take(boxes_v[pl.ds(f * BPW + chunk, _L)], lane)

            b_f = field(0)
            x1 = field(1) * _SCALE - 0.5
            y1 = field(2) * _SCALE - 0.5
            x2 = field(3) * _SCALE - 0.5
            y2 = field(4) * _SCALE - 0.5
            binw = (x2 - x1) * (1.0 / _P)
            binh = (y2 - y1) * (1.0 / _P)
            grid = (iota >> 1).astype(jnp.float32) + \
                ((iota & 1).astype(jnp.float32) + 0.5) * (1.0 / _S)
            xs = x1 + grid * binw
            ys = y1 + grid * binh

            def prep(v, L):
                valid = (v >= -1.0) & (v <= float(L))
                v = jnp.maximum(v, 0.0)
                low = jnp.minimum(v.astype(jnp.int32), L - 1)
                high = jnp.minimum(low + 1, L - 1)
                frac = jnp.where(low >= L - 1, 0.0, v - low.astype(jnp.float32))
                wlo = jnp.where(valid, (1.0 - frac) * (1.0 / _S), 0.0)
                whi = jnp.where(valid, frac * (1.0 / _S), 0.0)
                return low, high, wlo, whi

            yl, yh, wloy, whiy = prep(ys, H)
            xl, xh, wlox, whix = prep(xs, W)
            bbase = b_f.astype(jnp.int32) * (H * W)
            pf_v[0] = wloy
            pf_v[1] = whiy
            pf_v[2] = wlox
            pf_v[3] = whix
            pi_v[0] = bbase + yl * W
            pi_v[1] = bbase + yh * W
            pi_v[2] = xl
            pi_v[3] = xh

        def fire_gather(p, par):
            xlv = pi_v[2]
            xhv = pi_v[3]
            for i_off in range(_S):
                i = _S * p + i_off
                for yc in range(2):
                    ybase = _take(pi_v[yc], i)
                    for xc in range(2):
                        gg = (i_off * 2 + yc) * 2 + xc
                        vals = ybase + (xlv if xc == 0 else xhv)
                        plsc.store_scatter(
                            idx_v,
                            [jnp.full((_L,), par, jnp.int32), iota + gg * _KQ],
                            vals, mask=lane_mask)
            pltpu.async_copy(table_hbm.at[idx_v.at[par]], g_v.at[par], gsem)

        compute_params(0)
        fire_gather(0, 0)
        NSTEP = BPW * _P

        def step(s, carry):
            bi = s // _P
            p = s - bi * _P
            par = s & 1
            obuf = bi & 1

            wy = []
            for i_off in range(_S):
                i = _S * p + i_off
                wy.append((_bf16x2(_take(pf_v[0], i)), _bf16x2(_take(pf_v[1], i))))
            wlox = pf_v[2]
            whix = pf_v[3]

            @pl.when(p < _P - 1)
            def _():
                fire_gather(p + 1, 1 - par)

            @pl.when(jnp.logical_and(p == _P - 1, bi < BPW - 1))
            def _():
                compute_params(bi + 1)
                fire_gather(0, 1 - par)

            @pl.when(jnp.logical_and(p == 0, bi >= 2))
            def _():
                pltpu.make_async_copy(
                    o_v.at[obuf, pl.ds(0, PP)], out_hbm.at[wid * BPW + bi], osem).wait()

            pltpu.make_async_copy(table_hbm.at[idx_v.at[par]], g_v.at[par],
                                  gsem).wait()

            def q_body(q, carry3):
                terms = []
                for j_off in range(_S):
                    j = _S * q + j_off
                    wx = (_bf16x2(_take(wlox, j)), _bf16x2(_take(whix, j)))
                    for i_off in range(_S):
                        for yc in range(2):
                            for xc in range(2):
                                w32 = wy[i_off][yc] * wx[xc]
                                row = ((i_off * 2 + yc) * 2 + xc) * _KQ + j
                                terms.append((w32, row))
                for k in range(CW // _L):
                    prods = [w32 * plsc.bitcast(g_v[par, row, pl.ds(k * _L, _L)],
                                                jnp.bfloat16)
                             for (w32, row) in terms]
                    while len(prods) > 1:
                        prods = [prods[z] + prods[z + 1]
                                 for z in range(0, len(prods), 2)]
                    o_v[obuf, p * _P + q, pl.ds(k * _L, _L)] = \
                        plsc.bitcast(prods[0], jnp.int32)
                return carry3

            lax.fori_loop(0, _P, q_body, None)

            @pl.when(p == _P - 1)
            def _():
                pltpu.async_copy(o_v.at[obuf, pl.ds(0, PP)], out_hbm.at[wid * BPW + bi], osem)

            return carry

        lax.fori_loop(0, NSTEP, step, None)
        for tail in (BPW - 2, BPW - 1):
            pltpu.make_async_copy(o_v.at[tail & 1, pl.ds(0, PP)],
                                  out_hbm.at[wid * BPW + tail], osem).wait()

    return roi_sc


def _rne_bf16_bits(x):
    b = lax.bitcast_convert_type(x, jnp.uint32)
    b = b + jnp.uint32(0x7FFF) + ((b >> 16) & jnp.uint32(1))
    return b & jnp.uint32(0xFFFF0000)


def kernel(input, boxes):
    N, C, H, W = input.shape
    M = boxes.shape[0]
    lo = _rne_bf16_bits(input[:, :C // 2]) >> 16
    hi = _rne_bf16_bits(input[:, C // 2:])
    words = (lo | hi).astype(jnp.int32)
    table = jnp.transpose(words, (0, 2, 3, 1)).reshape(N * H * W, C // 2)
    roi_sc = _make_roi_kernel(N, C, H, W, M)
    out = roi_sc(table, jnp.transpose(boxes).reshape(-1))
    ow = lax.bitcast_convert_type(jnp.transpose(out, (0, 2, 1)), jnp.uint32)
    lo_f = lax.bitcast_convert_type(ow << 16, jnp.float32)
    hi_f = lax.bitcast_convert_type(ow & jnp.uint32(0xFFFF0000), jnp.float32)
    res = jnp.concatenate([lo_f, hi_f], axis=1)
    return res.reshape(M, C, _P, _P)


# speedup vs baseline: 8.8037x; 8.8037x over previous
import functools

import jax
import jax.numpy as jnp
from jax import lax
from jax.experimental import pallas as pl
from jax.experimental.pallas import tpu as pltpu
from jax.experimental.pallas import tpu_sc as plsc

_P = 7
_S = 2
_SCALE = 0.25
_L = 16
_NC = 2
_NS = 16
_NW = _NC * _NS


def _bf16x2(w):
    wi = plsc.bitcast(w, jnp.int32)
    wi = (wi + jnp.int32(0x8000)) & jnp.int32(-0x10000)
    pair = wi | lax.shift_right_logical(wi, 16)
    return plsc.bitcast(pair, jnp.bfloat16)


def _take(vec, i):
    idx = jnp.full((_L, 1), i, jnp.int32)
    dnums = lax.GatherDimensionNumbers(
        offset_dims=(), collapsed_slice_dims=(0,), start_index_map=(0,))
    return lax.gather(vec, idx, dnums, (1,),
                      mode=lax.GatherScatterMode.PROMISE_IN_BOUNDS)


@functools.lru_cache(maxsize=None)
def _make_roi_kernel(N, C, H, W, M):
    BPW = M // _NW
    PP = _P * _P
    CW = C // 2
    mesh = plsc.VectorSubcoreMesh(core_axis_name="c", subcore_axis_name="s")

    @functools.partial(
        pl.kernel,
        mesh=mesh,
        compiler_params=pltpu.CompilerParams(needs_layout_passes=False),
        out_type=jax.ShapeDtypeStruct((M, PP, CW), jnp.int32),
        scratch_types=[
            pltpu.VMEM((5 * BPW,), jnp.float32),
            pltpu.VMEM((2, 8 * _L), jnp.int32),
            pltpu.VMEM((2, 8 * _L, CW), jnp.int32),
            pltpu.VMEM((PP, CW), jnp.int32),
            pltpu.SemaphoreType.DMA,
        ],
    )
    def roi_sc(table_hbm, boxes_hbm, out_hbm, boxes_v, idx_v, g_v, o_v, sem):
        wid = lax.axis_index("s") * _NC + lax.axis_index("c")
        for f in range(5):
            pltpu.sync_copy(boxes_hbm.at[pl.ds(f * M + wid * BPW, BPW)],
                            boxes_v.at[pl.ds(f * BPW, BPW)])

        def box_body(bi, carry):
            chunk = (bi // _L) * _L
            lane = bi - chunk

            def field(f):
                return _take(boxes_v[pl.ds(f * BPW + chunk, _L)], lane)

            b_f = field(0)
            x1 = field(1) * _SCALE - 0.5
            y1 = field(2) * _SCALE - 0.5
            x2 = field(3) * _SCALE - 0.5
            y2 = field(4) * _SCALE - 0.5
            binw = (x2 - x1) * (1.0 / _P)
            binh = (y2 - y1) * (1.0 / _P)
            t = lax.iota(jnp.int32, _L)
            grid = (t >> 1).astype(jnp.float32) + ((t & 1).astype(jnp.float32) + 0.5) * (1.0 / _S)
            xs = x1 + grid * binw
            ys = y1 + grid * binh

            def prep(v, L):
                valid = (v >= -1.0) & (v <= float(L))
                v = jnp.maximum(v, 0.0)
                low = jnp.minimum(v.astype(jnp.int32), L - 1)
                high = jnp.minimum(low + 1, L - 1)
                frac = jnp.where(low >= L - 1, 0.0, v - low.astype(jnp.float32))
                wlo = jnp.where(valid, (1.0 - frac) * (1.0 / _S), 0.0)
                whi = jnp.where(valid, frac * (1.0 / _S), 0.0)
                return low, high, wlo, whi

            yl, yh, wloy, whiy = prep(ys, H)
            xl, xh, wlox, whix = prep(xs, W)
            bbase = b_f.astype(jnp.int32) * (H * W)

            def start_gather(p):
                par = p & 1
                for i_off in range(_S):
                    i = _S * p + i_off
                    for yc in range(2):
                        yv = _take(yl if yc == 0 else yh, i)
                        row_base = bbase + yv * W
                        for xc in range(2):
                            g = i_off * 4 + yc * 2 + xc
                            idx_v[par, pl.ds(g * _L, _L)] = row_base + (xl if xc == 0 else xh)
                pltpu.async_copy(table_hbm.at[idx_v.at[par]], g_v.at[par], sem)

            start_gather(0)

            def p_body(p, carry2):
                par = p & 1

                @pl.when(p < _P - 1)
                def _():
                    start_gather(p + 1)

                pltpu.make_async_copy(table_hbm.at[idx_v.at[par]], g_v.at[par], sem).wait()

                wy = []
                for i_off in range(_S):
                    i = _S * p + i_off
                    wy.append((_bf16x2(_take(wloy, i)), _bf16x2(_take(whiy, i))))

                def q_body(q, carry3):
                    terms = []
                    for j_off in range(_S):
                        j = _S * q + j_off
                        wx = (_bf16x2(_take(wlox, j)), _bf16x2(_take(whix, j)))
                        for i_off in range(_S):
                            for yc in range(2):
                                for xc in range(2):
                                    w32 = wy[i_off][yc] * wx[xc]
                                    row = (i_off * 4 + yc * 2 + xc) * _L + j
                                    terms.append((w32, row))
                    for k in range(CW // _L):
                        prods = [w32 * plsc.bitcast(g_v[par, row, pl.ds(k * _L, _L)],
                                                    jnp.bfloat16)
                                 for (w32, row) in terms]
                        while len(prods) > 1:
                            prods = [prods[z] + prods[z + 1]
                                     for z in range(0, len(prods), 2)]
                        o_v[p * _P + q, pl.ds(k * _L, _L)] = plsc.bitcast(prods[0], jnp.int32)
                    return carry3

                lax.fori_loop(0, _P, q_body, None)
                return carry2

            lax.fori_loop(0, _P, p_body, None)
            pltpu.sync_copy(o_v, out_hbm.at[wid * BPW + bi])
            return carry

        lax.fori_loop(0, BPW, box_body, None)

    return roi_sc


def _rne_bf16_bits(x):
    b = lax.bitcast_convert_type(x, jnp.uint32)
    b = b + jnp.uint32(0x7FFF) + ((b >> 16) & jnp.uint32(1))
    return b & jnp.uint32(0xFFFF0000)


def kernel(input, boxes):
    N, C, H, W = input.shape
    M = boxes.shape[0]
    lo = _rne_bf16_bits(input[:, :C // 2]) >> 16
    hi = _rne_bf16_bits(input[:, C // 2:])
    words = (lo | hi).astype(jnp.int32)
    table = jnp.transpose(words, (0, 2, 3, 1)).reshape(N * H * W, C // 2)
    roi_sc = _make_roi_kernel(N, C, H, W, M)
    out = roi_sc(table, jnp.transpose(boxes).reshape(-1))
    ow = lax.bitcast_convert_type(jnp.transpose(out, (0, 2, 1)), jnp.uint32)
    lo_f = lax.bitcast_convert_type(ow << 16, jnp.float32)
    hi_f = lax.bitcast_convert_type(ow & jnp.uint32(0xFFFF0000), jnp.float32)
    res = jnp.concatenate([lo_f, hi_f], axis=1)
    return res.reshape(M, C, _P, _P)
